# baseline (speedup 1.0000x reference)
import functools

import jax
import jax.numpy as jnp
from jax import lax
from jax.experimental import pallas as pl
from jax.experimental.pallas import tpu as pltpu
from jax.experimental.pallas import tpu_sc as plsc

N_NODES = 10000
N_EDGES = 640000
HALF = 128
CHUNK = 125
ROWS = N_EDGES // CHUNK
NC, NS = 2, 16
ROWS_PER_TILE = ROWS // NS
ROWS_PER_TILE_A = ROWS // (NC * NS)
NODE_SPLIT = 640

_MESH = plsc.VectorSubcoreMesh(
    core_axis_name="c", subcore_axis_name="s", num_cores=NC, num_subcores=NS
)


DEGW = 128
ZCHUNK = 80


@functools.partial(
    pl.kernel,
    out_type=jax.ShapeDtypeStruct((NC, N_NODES, DEGW), jnp.float32),
    mesh=_MESH,
    scratch_types=[
        pltpu.VMEM((ROWS_PER_TILE_A, CHUNK), jnp.int32),
        pltpu.VMEM((CHUNK, DEGW), jnp.float32),
        pltpu.VMEM((ZCHUNK, DEGW), jnp.float32),
        pltpu.VMEM_SHARED((N_NODES, DEGW), jnp.float32),
        pltpu.SemaphoreType.DMA,
    ],
    name="gnn_degree_sc",
)
def _deg_kernel(dst2d, ones_c, zdeg, deg2, idx_v, ones_v, zbuf_v, deg_sp, sem):
    c = lax.axis_index("c")
    s = lax.axis_index("s")
    nb = s * NODE_SPLIT
    nchunks = jnp.where(s < NS - 1, NODE_SPLIT // ZCHUNK, 400 // ZCHUNK)
    pltpu.sync_copy(ones_c, ones_v)
    pltpu.sync_copy(zdeg, zbuf_v)

    def zbody(k, carry):
        pltpu.sync_copy(zbuf_v, deg_sp.at[pl.ds(nb + ZCHUNK * k, ZCHUNK)])
        return carry

    lax.fori_loop(0, nchunks, zbody, 0)

    row0 = (c * NS + s) * ROWS_PER_TILE_A
    pltpu.sync_copy(dst2d.at[pl.ds(row0, ROWS_PER_TILE_A)], idx_v)
    plsc.subcore_barrier()

    def body(k, carry):
        ds = [pltpu.async_copy(ones_v, deg_sp.at[idx_v.at[4 * k + i]],
                               sem, add=True) for i in range(4)]
        for d in ds:
            d.wait()
        return carry

    lax.fori_loop(0, ROWS_PER_TILE_A // 4, body, 0)
    plsc.subcore_barrier()

    def dbody(k, carry):
        pltpu.sync_copy(deg_sp.at[pl.ds(nb + ZCHUNK * k, ZCHUNK)], zbuf_v)
        pltpu.sync_copy(zbuf_v, deg2.at[c, pl.ds(nb + ZCHUNK * k, ZCHUNK)])
        return carry

    lax.fori_loop(0, nchunks, dbody, 0)


_PREP_BLK = 1000


def _prep_body(deg2_ref, x_ref, dinv_ref, ys_ref, yf_ref):
    deg = deg2_ref[0, :, :1] + deg2_ref[1, :, :1] + 1.0
    dinv = lax.rsqrt(deg)
    dinv_ref[...] = dinv
    ys_ref[...] = x_ref[:, :HALF] * dinv
    yf_ref[...] = x_ref[:, HALF:] * dinv


_prep = pl.pallas_call(
    _prep_body,
    grid=(N_NODES // _PREP_BLK,),
    in_specs=[
        pl.BlockSpec((NC, _PREP_BLK, DEGW), lambda i: (0, i, 0)),
        pl.BlockSpec((_PREP_BLK, 2 * HALF), lambda i: (i, 0)),
    ],
    out_specs=[
        pl.BlockSpec((_PREP_BLK, 1), lambda i: (i, 0)),
        pl.BlockSpec((_PREP_BLK, HALF), lambda i: (i, 0)),
        pl.BlockSpec((_PREP_BLK, HALF), lambda i: (i, 0)),
    ],
    out_shape=[
        jax.ShapeDtypeStruct((N_NODES, 1), jnp.float32),
        jax.ShapeDtypeStruct((N_NODES, HALF), jnp.float32),
        jax.ShapeDtypeStruct((N_NODES, HALF), jnp.float32),
    ],
    name="gnn_prep_tc",
)


IDX_BLK = 64
ZCHUNK_C = ZCHUNK


@functools.partial(
    pl.kernel,
    out_type=jax.ShapeDtypeStruct((NC, N_NODES, HALF), jnp.float32),
    mesh=_MESH,
    scratch_types=[
        pltpu.VMEM((IDX_BLK, CHUNK), jnp.int32),
        pltpu.VMEM((IDX_BLK, CHUNK), jnp.int32),
        pltpu.VMEM((2, CHUNK, HALF), jnp.float32),
        pltpu.VMEM_SHARED((N_NODES, HALF), jnp.float32),
        pltpu.SemaphoreType.DMA,
        pltpu.SemaphoreType.DMA,
    ],
    name="gnn_edge_agg_sc",
)
def _agg_kernel(src2d, dst2d, ys, yf, zrow, acc2,
                src_v, dst_v, rows_v, acc_sp, gsem, ssem):
    c = lax.axis_index("c")
    s = lax.axis_index("s")
    nb = s * NODE_SPLIT
    nchunks = jnp.where(s < NS - 1, NODE_SPLIT // ZCHUNK_C, 400 // ZCHUNK_C)
    zbuf_v = rows_v.at[0, pl.ds(0, ZCHUNK_C)]

    pltpu.sync_copy(zrow, zbuf_v)

    def zbody(k, carry):
        pltpu.sync_copy(zbuf_v, acc_sp.at[pl.ds(nb + ZCHUNK_C * k, ZCHUNK_C)])
        return carry

    lax.fori_loop(0, nchunks, zbody, 0)
    plsc.subcore_barrier()

    def edge_loop(ytab):
        def outer(b, carry):
            r0 = s * ROWS_PER_TILE + b * IDX_BLK
            pltpu.sync_copy(src2d.at[pl.ds(r0, IDX_BLK)], src_v)
            pltpu.sync_copy(dst2d.at[pl.ds(r0, IDX_BLK)], dst_v)
            pltpu.async_copy(ytab.at[src_v.at[0]], rows_v.at[0], gsem).wait()

            def body(j, carry2):
                p = j % 2
                jn = jnp.minimum(j + 1, IDX_BLK - 1)
                dg = pltpu.async_copy(ytab.at[src_v.at[jn]],
                                      rows_v.at[1 - p], gsem)
                ds = pltpu.async_copy(rows_v.at[p], acc_sp.at[dst_v.at[j]],
                                      ssem, add=True)
                ds.wait()
                dg.wait()
                return carry2

            lax.fori_loop(0, IDX_BLK, body, 0)
            return carry

        lax.fori_loop(0, ROWS_PER_TILE // IDX_BLK, outer, 0)

    @pl.when(c == 0)
    def _():
        edge_loop(ys)

    @pl.when(c == 1)
    def _():
        edge_loop(yf)

    plsc.subcore_barrier()

    def dbody(k, carry):
        pltpu.sync_copy(acc_sp.at[pl.ds(nb + ZCHUNK_C * k, ZCHUNK_C)], zbuf_v)
        pltpu.sync_copy(zbuf_v, acc2.at[c, pl.ds(nb + ZCHUNK_C * k, ZCHUNK_C)])
        return carry

    lax.fori_loop(0, nchunks, dbody, 0)


_TAIL_BLK = 1000


def _tail_body(acc2_ref, x_ref, dinv_ref, Ws_ref, bs_ref, Wf_ref, bf_ref,
               Wfus_ref, bfus_ref, Wout_ref, bout_ref, o_ref, ssum, fsum):
    i = pl.program_id(0)
    dinv = dinv_ref[...]
    d2 = dinv * dinv
    aggs = dinv * acc2_ref[0] + d2 * x_ref[:, :HALF]
    aggf = dinv * acc2_ref[1] + d2 * x_ref[:, HALF:]
    hs = jnp.maximum(
        jnp.dot(aggs, Ws_ref[...], preferred_element_type=jnp.float32)
        + bs_ref[...], 0.0)
    hf = jnp.maximum(
        jnp.dot(aggf, Wf_ref[...], preferred_element_type=jnp.float32)
        + bf_ref[...], 0.0)
    ps = jnp.sum(hs, axis=0, keepdims=True)
    pf = jnp.sum(hf, axis=0, keepdims=True)

    @pl.when(i == 0)
    def _():
        ssum[...] = ps
        fsum[...] = pf

    @pl.when(i > 0)
    def _():
        ssum[...] += ps
        fsum[...] += pf

    @pl.when(i == pl.num_programs(0) - 1)
    def _():
        pooled = (
            jnp.dot(ssum[...], Wfus_ref[: 2 * HALF],
                    preferred_element_type=jnp.float32)
            + jnp.dot(fsum[...], Wfus_ref[2 * HALF:],
                      preferred_element_type=jnp.float32)
        ) / float(N_NODES) + bfus_ref[...]
        o_ref[...] = jnp.dot(pooled, Wout_ref[...],
                             preferred_element_type=jnp.float32) + bout_ref[...]


_tail = pl.pallas_call(
    _tail_body,
    grid=(N_NODES // _TAIL_BLK,),
    in_specs=[
        pl.BlockSpec((NC, _TAIL_BLK, HALF), lambda i: (0, i, 0)),
        pl.BlockSpec((_TAIL_BLK, 2 * HALF), lambda i: (i, 0)),
        pl.BlockSpec((_TAIL_BLK, 1), lambda i: (i, 0)),
        pl.BlockSpec((HALF, 2 * HALF), lambda i: (0, 0)),
        pl.BlockSpec((1, 2 * HALF), lambda i: (0, 0)),
        pl.BlockSpec((HALF, 2 * HALF), lambda i: (0, 0)),
        pl.BlockSpec((1, 2 * HALF), lambda i: (0, 0)),
        pl.BlockSpec((4 * HALF, 2 * HALF), lambda i: (0, 0)),
        pl.BlockSpec((1, 2 * HALF), lambda i: (0, 0)),
        pl.BlockSpec((2 * HALF, 1), lambda i: (0, 0)),
        pl.BlockSpec((1, 1), lambda i: (0, 0)),
    ],
    out_specs=pl.BlockSpec((1, 1), lambda i: (0, 0)),
    out_shape=jax.ShapeDtypeStruct((1, 1), jnp.float32),
    scratch_shapes=[
        pltpu.VMEM((1, 2 * HALF), jnp.float32),
        pltpu.VMEM((1, 2 * HALF), jnp.float32),
    ],
    name="gnn_tail_tc",
)


def kernel(x, edge_index, Ws, bs, Wf, bf, Wfus, bfus, Wout, bout):
    src2d = edge_index[0].reshape(ROWS, CHUNK)
    dst2d = edge_index[1].reshape(ROWS, CHUNK)
    ones_c = jnp.ones((CHUNK, DEGW), jnp.float32)
    zdeg = jnp.zeros((ZCHUNK, DEGW), jnp.float32)
    zrow = jnp.zeros((ZCHUNK_C, HALF), jnp.float32)

    deg2 = _deg_kernel(dst2d, ones_c, zdeg)
    dinv, ys, yf = _prep(deg2, x)
    acc2 = _agg_kernel(src2d, dst2d, ys, yf, zrow)
    o = _tail(acc2, x, dinv, Ws, bs.reshape(1, -1), Wf, bf.reshape(1, -1),
              Wfus, bfus.reshape(1, -1), Wout, bout.reshape(1, 1))
    return o.reshape(1)


# speedup vs baseline: 1.1792x; 1.1792x over previous
import functools

import jax
import jax.numpy as jnp
from jax import lax
from jax.experimental import pallas as pl
from jax.experimental.pallas import tpu as pltpu
from jax.experimental.pallas import tpu_sc as plsc

N_NODES = 10000
N_EDGES = 640000
HALF = 128
CHUNK = 100
ROWS = N_EDGES // CHUNK
NC, NS = 2, 16
ROWS_PER_TILE = ROWS // NS
ROWS_PER_TILE_A = ROWS // (NC * NS)
NODE_SPLIT = 640

_MESH = plsc.VectorSubcoreMesh(
    core_axis_name="c", subcore_axis_name="s", num_cores=NC, num_subcores=NS
)


DEGW = 128
ZCHUNK = 80


@functools.partial(
    pl.kernel,
    out_type=jax.ShapeDtypeStruct((NC, N_NODES, DEGW), jnp.float32),
    mesh=_MESH,
    scratch_types=[
        pltpu.VMEM((ROWS_PER_TILE_A, CHUNK), jnp.int32),
        pltpu.VMEM((CHUNK, DEGW), jnp.float32),
        pltpu.VMEM((ZCHUNK, DEGW), jnp.float32),
        pltpu.VMEM_SHARED((N_NODES, DEGW), jnp.float32),
        pltpu.SemaphoreType.DMA,
    ],
    name="gnn_degree_sc",
)
def _deg_kernel(dst2d, ones_c, zdeg, deg2, idx_v, ones_v, zbuf_v, deg_sp, sem):
    c = lax.axis_index("c")
    s = lax.axis_index("s")
    nb = s * NODE_SPLIT
    nchunks = jnp.where(s < NS - 1, NODE_SPLIT // ZCHUNK, 400 // ZCHUNK)
    pltpu.sync_copy(ones_c, ones_v)
    pltpu.sync_copy(zdeg, zbuf_v)

    def zbody(k, carry):
        pltpu.sync_copy(zbuf_v, deg_sp.at[pl.ds(nb + ZCHUNK * k, ZCHUNK)])
        return carry

    lax.fori_loop(0, nchunks, zbody, 0)

    row0 = (c * NS + s) * ROWS_PER_TILE_A
    pltpu.sync_copy(dst2d.at[pl.ds(row0, ROWS_PER_TILE_A)], idx_v)
    plsc.subcore_barrier()

    def body(k, carry):
        ds = [pltpu.async_copy(ones_v, deg_sp.at[idx_v.at[4 * k + i]],
                               sem, add=True) for i in range(4)]
        for d in ds:
            d.wait()
        return carry

    lax.fori_loop(0, ROWS_PER_TILE_A // 4, body, 0)
    plsc.subcore_barrier()

    def dbody(k, carry):
        pltpu.sync_copy(deg_sp.at[pl.ds(nb + ZCHUNK * k, ZCHUNK)], zbuf_v)
        pltpu.sync_copy(zbuf_v, deg2.at[c, pl.ds(nb + ZCHUNK * k, ZCHUNK)])
        return carry

    lax.fori_loop(0, nchunks, dbody, 0)


_PREP_BLK = 1000


def _prep_body(deg2_ref, x_ref, dinv_ref, ys_ref, yf_ref):
    deg = deg2_ref[0, :, :1] + deg2_ref[1, :, :1] + 1.0
    dinv = lax.rsqrt(deg)
    dinv_ref[...] = dinv
    ys_ref[...] = x_ref[:, :HALF] * dinv
    yf_ref[...] = x_ref[:, HALF:] * dinv


_prep = pl.pallas_call(
    _prep_body,
    grid=(N_NODES // _PREP_BLK,),
    in_specs=[
        pl.BlockSpec((NC, _PREP_BLK, DEGW), lambda i: (0, i, 0)),
        pl.BlockSpec((_PREP_BLK, 2 * HALF), lambda i: (i, 0)),
    ],
    out_specs=[
        pl.BlockSpec((_PREP_BLK, 1), lambda i: (i, 0)),
        pl.BlockSpec((_PREP_BLK, HALF), lambda i: (i, 0)),
        pl.BlockSpec((_PREP_BLK, HALF), lambda i: (i, 0)),
    ],
    out_shape=[
        jax.ShapeDtypeStruct((N_NODES, 1), jnp.float32),
        jax.ShapeDtypeStruct((N_NODES, HALF), jnp.float32),
        jax.ShapeDtypeStruct((N_NODES, HALF), jnp.float32),
    ],
    name="gnn_prep_tc",
)


IDX_BLK = 16
NRING = 3
ZCHUNK_C = ZCHUNK


@functools.partial(
    pl.kernel,
    out_type=jax.ShapeDtypeStruct((NC, N_NODES, HALF), jnp.float32),
    mesh=_MESH,
    scratch_types=[
        pltpu.VMEM((2, IDX_BLK, CHUNK), jnp.int32),
        pltpu.VMEM((2, IDX_BLK, CHUNK), jnp.int32),
        pltpu.VMEM((NRING, CHUNK, HALF), jnp.float32),
        pltpu.VMEM_SHARED((N_NODES, HALF), jnp.float32),
        pltpu.SemaphoreType.DMA,
        pltpu.SemaphoreType.DMA,
        pltpu.SemaphoreType.DMA,
        pltpu.SemaphoreType.DMA,
    ],
    name="gnn_edge_agg_sc",
)
def _agg_kernel(src2d, dst2d, ys, yf, zrow, acc2,
                src_v, dst_v, rows_v, acc_sp, gsem0, gsem1, ssem0, ssem1):
    c = lax.axis_index("c")
    s = lax.axis_index("s")
    nb = s * NODE_SPLIT
    nchunks = jnp.where(s < NS - 1, NODE_SPLIT // ZCHUNK_C, 400 // ZCHUNK_C)
    zbuf_v = rows_v.at[0, pl.ds(0, ZCHUNK_C)]

    pltpu.sync_copy(zrow, zbuf_v)

    def zbody(k, carry):
        pltpu.sync_copy(zbuf_v, acc_sp.at[pl.ds(nb + ZCHUNK_C * k, ZCHUNK_C)])
        return carry

    lax.fori_loop(0, nchunks, zbody, 0)
    plsc.subcore_barrier()

    def edge_loop(ytab):
        n = ROWS_PER_TILE
        row0 = s * ROWS_PER_TILE
        gsems = (gsem0, gsem1)
        ssems = (ssem0, ssem1)


        def src_row(q):
            return src_v.at[(q // IDX_BLK) % 2, q % IDX_BLK]

        def dst_row(q):
            return dst_v.at[(q // IDX_BLK) % 2, q % IDX_BLK]

        def start_gather(q, par):
            pltpu.async_copy(ytab.at[src_row(q)], rows_v.at[q % NRING],
                             gsems[par])

        def wait_gather(q, par):
            pltpu.make_async_copy(ytab.at[src_row(q)], rows_v.at[q % NRING],
                                  gsems[par]).wait()

        def start_scatter(q, par):
            pltpu.async_copy(rows_v.at[q % NRING], acc_sp.at[dst_row(q)],
                             ssems[par], add=True)

        def wait_scatter(q, par):
            pltpu.make_async_copy(rows_v.at[q % NRING], acc_sp.at[dst_row(q)],
                                  ssems[par]).wait()

        pltpu.sync_copy(src2d.at[pl.ds(row0, IDX_BLK)], src_v.at[0])
        pltpu.sync_copy(dst2d.at[pl.ds(row0, IDX_BLK)], dst_v.at[0])
        start_gather(0, 0)
        start_gather(1, 1)

        def body(k, carry):
            j0 = 2 * k
            j1 = j0 + 1
            b = j0 // IDX_BLK

            wait_gather(j0, 0)
            start_scatter(j0, 0)

            @pl.when(j0 > 0)
            def _():
                wait_scatter(j0 - 1, 1)

            @pl.when((j0 % IDX_BLK == 0) & (b + 1 < n // IDX_BLK))
            def _():
                r1 = row0 + (b + 1) * IDX_BLK
                pltpu.sync_copy(src2d.at[pl.ds(r1, IDX_BLK)],
                                src_v.at[(b + 1) % 2])
                pltpu.sync_copy(dst2d.at[pl.ds(r1, IDX_BLK)],
                                dst_v.at[(b + 1) % 2])

            @pl.when(j0 + 2 < n)
            def _():
                start_gather(j0 + 2, 0)

            wait_gather(j1, 1)
            start_scatter(j1, 1)
            wait_scatter(j0, 0)

            @pl.when(j1 + 2 < n)
            def _():
                start_gather(j1 + 2, 1)

            return carry

        lax.fori_loop(0, n // 2, body, 0)
        wait_scatter(n - 1, 1)

    @pl.when(c == 0)
    def _():
        edge_loop(ys)

    @pl.when(c == 1)
    def _():
        edge_loop(yf)

    plsc.subcore_barrier()

    def dbody(k, carry):
        pltpu.sync_copy(acc_sp.at[pl.ds(nb + ZCHUNK_C * k, ZCHUNK_C)], zbuf_v)
        pltpu.sync_copy(zbuf_v, acc2.at[c, pl.ds(nb + ZCHUNK_C * k, ZCHUNK_C)])
        return carry

    lax.fori_loop(0, nchunks, dbody, 0)


_TAIL_BLK = 1000


def _tail_body(acc2_ref, x_ref, dinv_ref, Ws_ref, bs_ref, Wf_ref, bf_ref,
               Wfus_ref, bfus_ref, Wout_ref, bout_ref, o_ref, ssum, fsum):
    i = pl.program_id(0)
    dinv = dinv_ref[...]
    d2 = dinv * dinv
    aggs = dinv * acc2_ref[0] + d2 * x_ref[:, :HALF]
    aggf = dinv * acc2_ref[1] + d2 * x_ref[:, HALF:]
    hs = jnp.maximum(
        jnp.dot(aggs, Ws_ref[...], preferred_element_type=jnp.float32)
        + bs_ref[...], 0.0)
    hf = jnp.maximum(
        jnp.dot(aggf, Wf_ref[...], preferred_element_type=jnp.float32)
        + bf_ref[...], 0.0)
    ps = jnp.sum(hs, axis=0, keepdims=True)
    pf = jnp.sum(hf, axis=0, keepdims=True)

    @pl.when(i == 0)
    def _():
        ssum[...] = ps
        fsum[...] = pf

    @pl.when(i > 0)
    def _():
        ssum[...] += ps
        fsum[...] += pf

    @pl.when(i == pl.num_programs(0) - 1)
    def _():
        pooled = (
            jnp.dot(ssum[...], Wfus_ref[: 2 * HALF],
                    preferred_element_type=jnp.float32)
            + jnp.dot(fsum[...], Wfus_ref[2 * HALF:],
                      preferred_element_type=jnp.float32)
        ) / float(N_NODES) + bfus_ref[...]
        o_ref[...] = jnp.dot(pooled, Wout_ref[...],
                             preferred_element_type=jnp.float32) + bout_ref[...]


_tail = pl.pallas_call(
    _tail_body,
    grid=(N_NODES // _TAIL_BLK,),
    in_specs=[
        pl.BlockSpec((NC, _TAIL_BLK, HALF), lambda i: (0, i, 0)),
        pl.BlockSpec((_TAIL_BLK, 2 * HALF), lambda i: (i, 0)),
        pl.BlockSpec((_TAIL_BLK, 1), lambda i: (i, 0)),
        pl.BlockSpec((HALF, 2 * HALF), lambda i: (0, 0)),
        pl.BlockSpec((1, 2 * HALF), lambda i: (0, 0)),
        pl.BlockSpec((HALF, 2 * HALF), lambda i: (0, 0)),
        pl.BlockSpec((1, 2 * HALF), lambda i: (0, 0)),
        pl.BlockSpec((4 * HALF, 2 * HALF), lambda i: (0, 0)),
        pl.BlockSpec((1, 2 * HALF), lambda i: (0, 0)),
        pl.BlockSpec((2 * HALF, 1), lambda i: (0, 0)),
        pl.BlockSpec((1, 1), lambda i: (0, 0)),
    ],
    out_specs=pl.BlockSpec((1, 1), lambda i: (0, 0)),
    out_shape=jax.ShapeDtypeStruct((1, 1), jnp.float32),
    scratch_shapes=[
        pltpu.VMEM((1, 2 * HALF), jnp.float32),
        pltpu.VMEM((1, 2 * HALF), jnp.float32),
    ],
    name="gnn_tail_tc",
)


def kernel(x, edge_index, Ws, bs, Wf, bf, Wfus, bfus, Wout, bout):
    src2d = edge_index[0].reshape(ROWS, CHUNK)
    dst2d = edge_index[1].reshape(ROWS, CHUNK)
    ones_c = jnp.ones((CHUNK, DEGW), jnp.float32)
    zdeg = jnp.zeros((ZCHUNK, DEGW), jnp.float32)
    zrow = jnp.zeros((ZCHUNK_C, HALF), jnp.float32)

    deg2 = _deg_kernel(dst2d, ones_c, zdeg)
    dinv, ys, yf = _prep(deg2, x)
    acc2 = _agg_kernel(src2d, dst2d, ys, yf, zrow)
    o = _tail(acc2, x, dinv, Ws, bs.reshape(1, -1), Wf, bf.reshape(1, -1),
              Wfus, bfus.reshape(1, -1), Wout, bout.reshape(1, 1))
    return o.reshape(1)
